# baseline (speedup 1.0000x reference)
import dataclasses
import functools

import jax
import jax.numpy as jnp
from jax import lax
from jax.experimental import pallas as pl
from jax.experimental.pallas import tpu as pltpu
from jax.experimental.pallas import tpu_sc as plsc

N_NODES = 10000
N_EDGES = 320000
D_IN = 128
D_HID = 64

NC = 2
NS = 16
NW = NC * NS
CHUNK = 128
STEPS = 80
E_PER_TILE = STEPS * CHUNK
PAD_PER_TILE = E_PER_TILE - N_EDGES // NW
N_ACC = N_NODES + PAD_PER_TILE
ZROWS = N_ACC // NS
VSTEPS = E_PER_TILE // 16

BLK = ZROWS
GRID = N_ACC // BLK

_MESH = plsc.VectorSubcoreMesh(core_axis_name="c", subcore_axis_name="s")
_SC_PARAMS = pltpu.CompilerParams(use_tc_tiling_on_sc=False)
_SC_REG_PARAMS = _SC_PARAMS
if "needs_layout_passes" in pltpu.CompilerParams.__dataclass_fields__:
    _SC_REG_PARAMS = dataclasses.replace(_SC_PARAMS, needs_layout_passes=False)


def _sc_segment_sum_rows(table, src3d, dst3d, zrows):
    @functools.partial(
        pl.kernel,
        out_type=jax.ShapeDtypeStruct((NC, NS, ZROWS, D_HID), jnp.float32),
        mesh=_MESH,
        compiler_params=_SC_PARAMS,
        scratch_types=[
            pltpu.VMEM((STEPS, CHUNK), jnp.int32),
            pltpu.VMEM((STEPS, CHUNK), jnp.int32),
            pltpu.VMEM((CHUNK, D_HID), jnp.float32),
            pltpu.VMEM((CHUNK, D_HID), jnp.float32),
            pltpu.VMEM_SHARED((N_ACC, D_HID), jnp.float32),
            pltpu.SemaphoreType.DMA,
            pltpu.SemaphoreType.DMA,
        ],
    )
    def k(table_hbm, src_hbm, dst_hbm, z_hbm, out_hbm,
          src_v, dst_v, rows0, rows1, acc_sh, sem0, sem1):
        c = lax.axis_index("c")
        s = lax.axis_index("s")
        wid = c * NS + s
        pltpu.sync_copy(z_hbm, acc_sh.at[pl.ds(s * ZROWS, ZROWS)])
        pltpu.sync_copy(src_hbm.at[wid], src_v)
        pltpu.sync_copy(dst_hbm.at[wid], dst_v)
        plsc.subcore_barrier()

        pltpu.async_copy(table_hbm.at[src_v.at[0]], rows0, sem0)

        @pl.loop(0, STEPS, step=2)
        def _(i):
            pltpu.async_copy(table_hbm.at[src_v.at[i + 1]], rows1, sem1)
            pltpu.make_async_copy(table_hbm.at[src_v.at[i]], rows0, sem0).wait()
            pltpu.sync_copy(rows0, acc_sh.at[dst_v.at[i]], add=True)

            @pl.when(i + 2 < STEPS)
            def _():
                pltpu.async_copy(table_hbm.at[src_v.at[i + 2]], rows0, sem0)

            pltpu.make_async_copy(table_hbm.at[src_v.at[i + 1]], rows1, sem1).wait()
            pltpu.sync_copy(rows1, acc_sh.at[dst_v.at[i + 1]], add=True)

        plsc.subcore_barrier()
        pltpu.sync_copy(acc_sh.at[pl.ds(s * ZROWS, ZROWS)], out_hbm.at[c, s])

    return k(table, src3d, dst3d, zrows)


def _sc_degree(dstf, zcol):
    @functools.partial(
        pl.kernel,
        out_type=jax.ShapeDtypeStruct((NC, NS, N_ACC), jnp.float32),
        mesh=_MESH,
        compiler_params=_SC_REG_PARAMS,
        scratch_types=[
            pltpu.VMEM((E_PER_TILE,), jnp.int32),
            pltpu.VMEM((N_ACC,), jnp.float32),
        ],
    )
    def k(dst_hbm, z_hbm, out_hbm, dst_v, acc_v):
        c = lax.axis_index("c")
        s = lax.axis_index("s")
        wid = c * NS + s
        pltpu.sync_copy(z_hbm, acc_v)
        pltpu.sync_copy(dst_hbm.at[wid], dst_v)
        one16 = jnp.ones((16,), jnp.float32)

        @pl.loop(0, VSTEPS)
        def _(i):
            didx = dst_v[pl.ds(i * 16, 16)]
            plsc.addupdate_scatter(acc_v, [didx], one16)

        pltpu.sync_copy(acc_v, out_hbm.at[c, s])

    return k(dstf, zcol)


def _sc_segment_sum_scalar(colrow, srcf, dstf, zcol):
    @functools.partial(
        pl.kernel,
        out_type=jax.ShapeDtypeStruct((NC, NS, N_ACC), jnp.float32),
        mesh=_MESH,
        compiler_params=_SC_REG_PARAMS,
        scratch_types=[
            pltpu.VMEM((N_ACC,), jnp.float32),
            pltpu.VMEM((E_PER_TILE,), jnp.int32),
            pltpu.VMEM((E_PER_TILE,), jnp.int32),
            pltpu.VMEM((N_ACC,), jnp.float32),
        ],
    )
    def k(col_hbm, src_hbm, dst_hbm, z_hbm, out_hbm, col_v, src_v, dst_v, acc_v):
        c = lax.axis_index("c")
        s = lax.axis_index("s")
        wid = c * NS + s
        pltpu.sync_copy(z_hbm, acc_v)
        pltpu.sync_copy(col_hbm.at[0], col_v)
        pltpu.sync_copy(src_hbm.at[wid], src_v)
        pltpu.sync_copy(dst_hbm.at[wid], dst_v)

        @pl.loop(0, VSTEPS)
        def _(i):
            sidx = src_v[pl.ds(i * 16, 16)]
            didx = dst_v[pl.ds(i * 16, 16)]
            vals = plsc.load_gather(col_v, [sidx])
            plsc.addupdate_scatter(acc_v, [didx], vals)

        pltpu.sync_copy(acc_v, out_hbm.at[c, s])

    return k(colrow, srcf, dstf, zcol)


def _tc_project1(xpad, wcat):
    def body(x_ref, w_ref, p_ref, r_ref):
        xw = jnp.dot(x_ref[...], w_ref[...], preferred_element_type=jnp.float32)
        p_ref[...] = xw[:, :D_HID]
        r_ref[...] = xw[:, D_HID:]

    return pl.pallas_call(
        body,
        grid=(GRID,),
        in_specs=[
            pl.BlockSpec((BLK, D_IN), lambda i: (i, 0)),
            pl.BlockSpec((D_IN, 2 * D_HID), lambda i: (0, 0)),
        ],
        out_specs=[
            pl.BlockSpec((BLK, D_HID), lambda i: (i, 0)),
            pl.BlockSpec((BLK, D_HID), lambda i: (i, 0)),
        ],
        out_shape=[
            jax.ShapeDtypeStruct((N_ACC, D_HID), jnp.float32),
            jax.ShapeDtypeStruct((N_ACC, D_HID), jnp.float32),
        ],
    )(xpad, wcat)


def _tc_middle(agg, degp, r1, b1row, w2cat):
    def body(a_ref, d_ref, r1_ref, b1_ref, w_ref, q_ref, r2_ref):
        a = a_ref[0, 0] + a_ref[1, 0]
        deg = jnp.sum(d_ref[...], axis=(0, 1))
        degc = jnp.maximum(deg, 1.0).reshape(BLK, 1)
        h = jnp.maximum(a / degc + b1_ref[...] + r1_ref[...], 0.0)
        qr = jnp.dot(h, w_ref[...], preferred_element_type=jnp.float32)
        q_ref[...] = qr[:, 0].reshape(1, BLK)
        r2_ref[...] = qr[:, 1].reshape(1, BLK)

    return pl.pallas_call(
        body,
        grid=(GRID,),
        in_specs=[
            pl.BlockSpec((NC, 1, BLK, D_HID), lambda i: (0, i, 0, 0)),
            pl.BlockSpec((NC, NS, BLK), lambda i: (0, 0, i)),
            pl.BlockSpec((BLK, D_HID), lambda i: (i, 0)),
            pl.BlockSpec((1, D_HID), lambda i: (0, 0)),
            pl.BlockSpec((D_HID, 2), lambda i: (0, 0)),
        ],
        out_specs=[
            pl.BlockSpec((1, BLK), lambda i: (0, i)),
            pl.BlockSpec((1, BLK), lambda i: (0, i)),
        ],
        out_shape=[
            jax.ShapeDtypeStruct((1, N_ACC), jnp.float32),
            jax.ShapeDtypeStruct((1, N_ACC), jnp.float32),
        ],
    )(agg, degp, r1, b1row, w2cat)


def _tc_final(agg2, degp, r2row, b2row):
    def body(a_ref, d_ref, r2_ref, b2_ref, o_ref):
        ssum = jnp.sum(a_ref[...], axis=(0, 1))
        deg = jnp.sum(d_ref[...], axis=(0, 1))
        o = ssum / jnp.maximum(deg, 1.0) + b2_ref[0, 0] + r2_ref[0, :]
        o_ref[...] = o.reshape(BLK, 1)

    return pl.pallas_call(
        body,
        grid=(GRID,),
        in_specs=[
            pl.BlockSpec((NC, NS, BLK), lambda i: (0, 0, i)),
            pl.BlockSpec((NC, NS, BLK), lambda i: (0, 0, i)),
            pl.BlockSpec((1, BLK), lambda i: (0, i)),
            pl.BlockSpec((1, 1), lambda i: (0, 0)),
        ],
        out_specs=pl.BlockSpec((BLK, 1), lambda i: (i, 0)),
        out_shape=jax.ShapeDtypeStruct((N_NODES, 1), jnp.float32),
    )(agg2, degp, r2row, b2row)


def kernel(x, edge_index, W1_l, b1, W1_r, W2_l, b2, W2_r):
    src = edge_index[0].astype(jnp.int32).reshape(NW, N_EDGES // NW)
    dst = edge_index[1].astype(jnp.int32).reshape(NW, N_EDGES // NW)
    srcf = jnp.pad(src, ((0, 0), (0, PAD_PER_TILE)))
    pad_dst = jnp.broadcast_to(
        N_NODES + jnp.arange(PAD_PER_TILE, dtype=jnp.int32), (NW, PAD_PER_TILE))
    dstf = jnp.concatenate([dst, pad_dst], axis=1)
    src3d = srcf.reshape(NW, STEPS, CHUNK)
    dst3d = dstf.reshape(NW, STEPS, CHUNK)

    xpad = jnp.pad(x, ((0, N_ACC - N_NODES), (0, 0)))
    wcat = jnp.concatenate([W1_l, W1_r], axis=1)
    w2cat = jnp.concatenate([W2_l, W2_r], axis=1)
    zcol = jnp.zeros((N_ACC,), jnp.float32)

    p_tab, r1 = _tc_project1(xpad, wcat)
    degp = _sc_degree(dstf, zcol)
    agg1 = _sc_segment_sum_rows(p_tab, src3d, dst3d,
                                jnp.zeros((ZROWS, D_HID), jnp.float32))
    q_row, r2_row = _tc_middle(agg1, degp, r1, b1.reshape(1, D_HID), w2cat)
    agg2 = _sc_segment_sum_scalar(q_row, srcf, dstf, zcol)
    return _tc_final(agg2, degp, r2_row, b2.reshape(1, 1))


# speedup vs baseline: 1.7247x; 1.7247x over previous
import dataclasses
import functools

import jax
import jax.numpy as jnp
from jax import lax
from jax.experimental import pallas as pl
from jax.experimental.pallas import tpu as pltpu
from jax.experimental.pallas import tpu_sc as plsc

N_NODES = 10000
N_EDGES = 320000
D_IN = 128
D_HID = 64

NC = 2
NS = 16
NW = NC * NS
CHUNK = 128
STEPS = 80
E_PER_TILE = STEPS * CHUNK
PAD_PER_TILE = E_PER_TILE - N_EDGES // NW
N_ACC = N_NODES + PAD_PER_TILE
ZROWS = N_ACC // NS
VSTEPS = E_PER_TILE // 16

BLK = ZROWS
GRID = N_ACC // BLK

_MESH = plsc.VectorSubcoreMesh(core_axis_name="c", subcore_axis_name="s")
_SC_PARAMS = pltpu.CompilerParams(use_tc_tiling_on_sc=False)
_SC_REG_PARAMS = _SC_PARAMS
if "needs_layout_passes" in pltpu.CompilerParams.__dataclass_fields__:
    _SC_REG_PARAMS = dataclasses.replace(_SC_PARAMS, needs_layout_passes=False)


def _sc_segment_sum_rows(table, src3d, dst3d, zrows):
    @functools.partial(
        pl.kernel,
        out_type=jax.ShapeDtypeStruct((NC, NS, ZROWS, D_HID), jnp.float32),
        mesh=_MESH,
        compiler_params=_SC_PARAMS,
        scratch_types=[
            pltpu.VMEM((STEPS, CHUNK), jnp.int32),
            pltpu.VMEM((STEPS, CHUNK), jnp.int32),
            pltpu.VMEM((CHUNK, D_HID), jnp.float32),
            pltpu.VMEM((CHUNK, D_HID), jnp.float32),
            pltpu.VMEM_SHARED((N_ACC, D_HID), jnp.float32),
            pltpu.SemaphoreType.DMA,
            pltpu.SemaphoreType.DMA,
        ],
    )
    def k(table_hbm, src_hbm, dst_hbm, z_hbm, out_hbm,
          src_v, dst_v, rows0, rows1, acc_sh, sem0, sem1):
        c = lax.axis_index("c")
        s = lax.axis_index("s")
        wid = c * NS + s
        pltpu.sync_copy(z_hbm, acc_sh.at[pl.ds(s * ZROWS, ZROWS)])
        pltpu.sync_copy(src_hbm.at[wid], src_v)
        pltpu.sync_copy(dst_hbm.at[wid], dst_v)
        plsc.subcore_barrier()

        pltpu.async_copy(table_hbm.at[src_v.at[0]], rows0, sem0)

        @pl.loop(0, STEPS, step=2)
        def _(i):
            pltpu.async_copy(table_hbm.at[src_v.at[i + 1]], rows1, sem1)
            pltpu.make_async_copy(table_hbm.at[src_v.at[i]], rows0, sem0).wait()
            pltpu.sync_copy(rows0, acc_sh.at[dst_v.at[i]], add=True)

            @pl.when(i + 2 < STEPS)
            def _():
                pltpu.async_copy(table_hbm.at[src_v.at[i + 2]], rows0, sem0)

            pltpu.make_async_copy(table_hbm.at[src_v.at[i + 1]], rows1, sem1).wait()
            pltpu.sync_copy(rows1, acc_sh.at[dst_v.at[i + 1]], add=True)

        plsc.subcore_barrier()
        pltpu.sync_copy(acc_sh.at[pl.ds(s * ZROWS, ZROWS)], out_hbm.at[c, s])

    return k(table, src3d, dst3d, zrows)


def _sc_degree(dstf, zcol):
    @functools.partial(
        pl.kernel,
        out_type=jax.ShapeDtypeStruct((NC, NS, N_ACC), jnp.float32),
        mesh=_MESH,
        compiler_params=_SC_REG_PARAMS,
        scratch_types=[
            pltpu.VMEM((E_PER_TILE,), jnp.int32),
            pltpu.VMEM((N_ACC,), jnp.float32),
        ],
    )
    def k(dst_hbm, z_hbm, out_hbm, dst_v, acc_v):
        c = lax.axis_index("c")
        s = lax.axis_index("s")
        wid = c * NS + s
        pltpu.sync_copy(z_hbm, acc_v)
        pltpu.sync_copy(dst_hbm.at[wid], dst_v)
        one16 = jnp.ones((16,), jnp.float32)

        @pl.loop(0, VSTEPS)
        def _(i):
            didx = dst_v[pl.ds(i * 16, 16)]
            plsc.addupdate_scatter(acc_v, [didx], one16)

        pltpu.sync_copy(acc_v, out_hbm.at[c, s])

    return k(dstf, zcol)


def _sc_segment_sum_scalar(colrow, srcf, dstf, zcol):
    @functools.partial(
        pl.kernel,
        out_type=jax.ShapeDtypeStruct((NC, NS, N_ACC), jnp.float32),
        mesh=_MESH,
        compiler_params=_SC_REG_PARAMS,
        scratch_types=[
            pltpu.VMEM((N_ACC,), jnp.float32),
            pltpu.VMEM((E_PER_TILE,), jnp.int32),
            pltpu.VMEM((E_PER_TILE,), jnp.int32),
            pltpu.VMEM((N_ACC,), jnp.float32),
        ],
    )
    def k(col_hbm, src_hbm, dst_hbm, z_hbm, out_hbm, col_v, src_v, dst_v, acc_v):
        c = lax.axis_index("c")
        s = lax.axis_index("s")
        wid = c * NS + s
        pltpu.sync_copy(z_hbm, acc_v)
        pltpu.sync_copy(col_hbm.at[0], col_v)
        pltpu.sync_copy(src_hbm.at[wid], src_v)
        pltpu.sync_copy(dst_hbm.at[wid], dst_v)

        @pl.loop(0, VSTEPS)
        def _(i):
            sidx = src_v[pl.ds(i * 16, 16)]
            didx = dst_v[pl.ds(i * 16, 16)]
            vals = plsc.load_gather(col_v, [sidx])
            plsc.addupdate_scatter(acc_v, [didx], vals)

        pltpu.sync_copy(acc_v, out_hbm.at[c, s])

    return k(colrow, srcf, dstf, zcol)


def _tc_project1(xpad, wcat):
    def body(x_ref, w_ref, p_ref, r_ref):
        xw = jnp.dot(x_ref[...], w_ref[...], preferred_element_type=jnp.float32)
        p_ref[...] = xw[:, :D_HID]
        r_ref[...] = xw[:, D_HID:]

    return pl.pallas_call(
        body,
        grid=(GRID,),
        in_specs=[
            pl.BlockSpec((BLK, D_IN), lambda i: (i, 0)),
            pl.BlockSpec((D_IN, 2 * D_HID), lambda i: (0, 0)),
        ],
        out_specs=[
            pl.BlockSpec((BLK, D_HID), lambda i: (i, 0)),
            pl.BlockSpec((BLK, D_HID), lambda i: (i, 0)),
        ],
        out_shape=[
            jax.ShapeDtypeStruct((N_ACC, D_HID), jnp.float32),
            jax.ShapeDtypeStruct((N_ACC, D_HID), jnp.float32),
        ],
    )(xpad, wcat)


def _tc_middle(agg, degp, r1, b1row, w2cat):
    def body(a_ref, d_ref, r1_ref, b1_ref, w_ref, q_ref, r2_ref):
        a = a_ref[0, 0] + a_ref[1, 0]
        deg = jnp.sum(d_ref[...], axis=(0, 1))
        degc = jnp.maximum(deg, 1.0).reshape(BLK, 1)
        h = jnp.maximum(a / degc + b1_ref[...] + r1_ref[...], 0.0)
        qr = jnp.dot(h, w_ref[...], preferred_element_type=jnp.float32)
        q_ref[...] = qr[:, 0].reshape(1, BLK)
        r2_ref[...] = qr[:, 1].reshape(1, BLK)

    return pl.pallas_call(
        body,
        grid=(GRID,),
        in_specs=[
            pl.BlockSpec((NC, 1, BLK, D_HID), lambda i: (0, i, 0, 0)),
            pl.BlockSpec((NC, NS, BLK), lambda i: (0, 0, i)),
            pl.BlockSpec((BLK, D_HID), lambda i: (i, 0)),
            pl.BlockSpec((1, D_HID), lambda i: (0, 0)),
            pl.BlockSpec((D_HID, 2), lambda i: (0, 0)),
        ],
        out_specs=[
            pl.BlockSpec((1, BLK), lambda i: (0, i)),
            pl.BlockSpec((1, BLK), lambda i: (0, i)),
        ],
        out_shape=[
            jax.ShapeDtypeStruct((1, N_ACC), jnp.float32),
            jax.ShapeDtypeStruct((1, N_ACC), jnp.float32),
        ],
    )(agg, degp, r1, b1row, w2cat)


def _tc_final(agg2, degp, r2row, b2row):
    def body(a_ref, d_ref, r2_ref, b2_ref, o_ref):
        ssum = jnp.sum(a_ref[...], axis=(0, 1))
        deg = jnp.sum(d_ref[...], axis=(0, 1))
        o = ssum / jnp.maximum(deg, 1.0) + b2_ref[0, 0] + r2_ref[0, :]
        o_ref[...] = o.reshape(BLK, 1)

    return pl.pallas_call(
        body,
        grid=(GRID,),
        in_specs=[
            pl.BlockSpec((NC, NS, BLK), lambda i: (0, 0, i)),
            pl.BlockSpec((NC, NS, BLK), lambda i: (0, 0, i)),
            pl.BlockSpec((1, BLK), lambda i: (0, i)),
            pl.BlockSpec((1, 1), lambda i: (0, 0)),
        ],
        out_specs=pl.BlockSpec((BLK, 1), lambda i: (i, 0)),
        out_shape=jax.ShapeDtypeStruct((N_NODES, 1), jnp.float32),
    )(agg2, degp, r2row, b2row)


def kernel(x, edge_index, W1_l, b1, W1_r, W2_l, b2, W2_r):
    src = edge_index[0].astype(jnp.int32).reshape(NW, N_EDGES // NW)
    dst = edge_index[1].astype(jnp.int32).reshape(NW, N_EDGES // NW)
    pad_src = jnp.broadcast_to(jnp.arange(PAD_PER_TILE, dtype=jnp.int32),
                               (NW, PAD_PER_TILE))
    srcf = jnp.concatenate([src, pad_src], axis=1)
    pad_dst = jnp.broadcast_to(
        N_NODES + jnp.arange(PAD_PER_TILE, dtype=jnp.int32), (NW, PAD_PER_TILE))
    dstf = jnp.concatenate([dst, pad_dst], axis=1)
    src3d = srcf.reshape(NW, STEPS, CHUNK)
    dst3d = dstf.reshape(NW, STEPS, CHUNK)

    xpad = jnp.pad(x, ((0, N_ACC - N_NODES), (0, 0)))
    wcat = jnp.concatenate([W1_l, W1_r], axis=1)
    w2cat = jnp.concatenate([W2_l, W2_r], axis=1)
    zcol = jnp.zeros((N_ACC,), jnp.float32)

    p_tab, r1 = _tc_project1(xpad, wcat)
    degp = _sc_degree(dstf, zcol)
    agg1 = _sc_segment_sum_rows(p_tab, src3d, dst3d,
                                jnp.zeros((ZROWS, D_HID), jnp.float32))
    q_row, r2_row = _tc_middle(agg1, degp, r1, b1.reshape(1, D_HID), w2cat)
    agg2 = _sc_segment_sum_scalar(q_row, srcf, dstf, zcol)
    return _tc_final(agg2, degp, r2_row, b2.reshape(1, 1))


# speedup vs baseline: 1.7706x; 1.0266x over previous
import dataclasses
import functools

import jax
import jax.numpy as jnp
from jax import lax
from jax.experimental import pallas as pl
from jax.experimental.pallas import tpu as pltpu
from jax.experimental.pallas import tpu_sc as plsc

N_NODES = 10000
N_EDGES = 320000
D_IN = 128
D_HID = 64

NC = 2
NS = 16
NW = NC * NS
CHUNK = 128
STEPS = 80
E_PER_TILE = STEPS * CHUNK
PAD_PER_TILE = E_PER_TILE - N_EDGES // NW
N_ACC = N_NODES + PAD_PER_TILE
ZROWS = N_ACC // NS
VSTEPS = E_PER_TILE // 16

BLK = ZROWS
GRID = N_ACC // BLK

_MESH = plsc.VectorSubcoreMesh(core_axis_name="c", subcore_axis_name="s")
_SC_PARAMS = pltpu.CompilerParams(use_tc_tiling_on_sc=False)
_SC_REG_PARAMS = _SC_PARAMS
if "needs_layout_passes" in pltpu.CompilerParams.__dataclass_fields__:
    _SC_REG_PARAMS = dataclasses.replace(_SC_PARAMS, needs_layout_passes=False)


def _sc_segment_sum_rows(table, src3d, dst3d, zrows, degp):
    @functools.partial(
        pl.kernel,
        out_type=jax.ShapeDtypeStruct((NC, NS, ZROWS, D_HID), jnp.float32),
        mesh=_MESH,
        compiler_params=_SC_PARAMS,
        scratch_types=[
            pltpu.VMEM((STEPS, CHUNK), jnp.int32),
            pltpu.VMEM((STEPS, CHUNK), jnp.int32),
            pltpu.VMEM((CHUNK, D_HID), jnp.float32),
            pltpu.VMEM((CHUNK, D_HID), jnp.float32),
            pltpu.VMEM_SHARED((N_ACC, D_HID), jnp.float32),
            pltpu.SemaphoreType.DMA,
            pltpu.SemaphoreType.DMA,
        ],
    )
    def k(table_hbm, src_hbm, dst_hbm, z_hbm, deg_hbm, out_hbm,
          src_v, dst_v, rows0, rows1, acc_sh, sem0, sem1):
        del deg_hbm
        c = lax.axis_index("c")
        s = lax.axis_index("s")
        wid = c * NS + s
        pltpu.sync_copy(z_hbm, acc_sh.at[pl.ds(s * ZROWS, ZROWS)])
        pltpu.sync_copy(src_hbm.at[wid], src_v)
        pltpu.sync_copy(dst_hbm.at[wid], dst_v)
        plsc.subcore_barrier()

        pltpu.async_copy(table_hbm.at[src_v.at[0]], rows0, sem0)

        @pl.loop(0, STEPS, step=2)
        def _(i):
            pltpu.async_copy(table_hbm.at[src_v.at[i + 1]], rows1, sem1)
            pltpu.make_async_copy(table_hbm.at[src_v.at[i]], rows0, sem0).wait()
            pltpu.sync_copy(rows0, acc_sh.at[dst_v.at[i]], add=True)

            @pl.when(i + 2 < STEPS)
            def _():
                pltpu.async_copy(table_hbm.at[src_v.at[i + 2]], rows0, sem0)

            pltpu.make_async_copy(table_hbm.at[src_v.at[i + 1]], rows1, sem1).wait()
            pltpu.sync_copy(rows1, acc_sh.at[dst_v.at[i + 1]], add=True)

        plsc.subcore_barrier()
        pltpu.sync_copy(acc_sh.at[pl.ds(s * ZROWS, ZROWS)], out_hbm.at[c, s])

    return k(table, src3d, dst3d, zrows, degp)


def _sc_degree(dstf, zcol):
    @functools.partial(
        pl.kernel,
        out_type=jax.ShapeDtypeStruct((NC, NS, N_ACC), jnp.float32),
        mesh=_MESH,
        compiler_params=_SC_REG_PARAMS,
        scratch_types=[
            pltpu.VMEM((E_PER_TILE,), jnp.int32),
            pltpu.VMEM((N_ACC,), jnp.float32),
        ],
    )
    def k(dst_hbm, z_hbm, out_hbm, dst_v, acc_v):
        c = lax.axis_index("c")
        s = lax.axis_index("s")
        wid = c * NS + s
        pltpu.sync_copy(z_hbm, acc_v)
        pltpu.sync_copy(dst_hbm.at[wid], dst_v)
        one16 = jnp.ones((16,), jnp.float32)

        @pl.loop(0, VSTEPS)
        def _(i):
            didx = dst_v[pl.ds(i * 16, 16)]
            plsc.addupdate_scatter(acc_v, [didx], one16)

        pltpu.sync_copy(acc_v, out_hbm.at[c, s])

    return k(dstf, zcol)


def _sc_segment_sum_scalar(colrow, srcf, dstf, zcol):
    @functools.partial(
        pl.kernel,
        out_type=jax.ShapeDtypeStruct((NC, NS, N_ACC), jnp.float32),
        mesh=_MESH,
        compiler_params=_SC_REG_PARAMS,
        scratch_types=[
            pltpu.VMEM((N_ACC,), jnp.float32),
            pltpu.VMEM((E_PER_TILE,), jnp.int32),
            pltpu.VMEM((E_PER_TILE,), jnp.int32),
            pltpu.VMEM((N_ACC,), jnp.float32),
        ],
    )
    def k(col_hbm, src_hbm, dst_hbm, z_hbm, out_hbm, col_v, src_v, dst_v, acc_v):
        c = lax.axis_index("c")
        s = lax.axis_index("s")
        wid = c * NS + s
        pltpu.sync_copy(z_hbm, acc_v)
        pltpu.sync_copy(col_hbm.at[0], col_v)
        pltpu.sync_copy(src_hbm.at[wid], src_v)
        pltpu.sync_copy(dst_hbm.at[wid], dst_v)

        @pl.loop(0, VSTEPS)
        def _(i):
            sidx = src_v[pl.ds(i * 16, 16)]
            didx = dst_v[pl.ds(i * 16, 16)]
            vals = plsc.load_gather(col_v, [sidx])
            plsc.addupdate_scatter(acc_v, [didx], vals)

        pltpu.sync_copy(acc_v, out_hbm.at[c, s])

    return k(colrow, srcf, dstf, zcol)


def _tc_project1(xpad, wcat):
    def body(x_ref, w_ref, p_ref, r_ref):
        xw = jnp.dot(x_ref[...], w_ref[...], preferred_element_type=jnp.float32)
        p_ref[...] = xw[:, :D_HID]
        r_ref[...] = xw[:, D_HID:]

    return pl.pallas_call(
        body,
        grid=(GRID,),
        in_specs=[
            pl.BlockSpec((BLK, D_IN), lambda i: (i, 0)),
            pl.BlockSpec((D_IN, 2 * D_HID), lambda i: (0, 0)),
        ],
        out_specs=[
            pl.BlockSpec((BLK, D_HID), lambda i: (i, 0)),
            pl.BlockSpec((BLK, D_HID), lambda i: (i, 0)),
        ],
        out_shape=[
            jax.ShapeDtypeStruct((N_ACC, D_HID), jnp.float32),
            jax.ShapeDtypeStruct((N_ACC, D_HID), jnp.float32),
        ],
    )(xpad, wcat)


def _tc_middle(agg, degp, r1, b1row, w2cat):
    def body(a_ref, d_ref, r1_ref, b1_ref, w_ref, q_ref, r2_ref):
        a = a_ref[0, 0] + a_ref[1, 0]
        deg = jnp.sum(d_ref[...], axis=(0, 1))
        degc = jnp.maximum(deg, 1.0).reshape(BLK, 1)
        h = jnp.maximum(a / degc + b1_ref[...] + r1_ref[...], 0.0)
        qr = jnp.dot(h, w_ref[...], preferred_element_type=jnp.float32)
        q_ref[...] = qr[:, 0].reshape(1, BLK)
        r2_ref[...] = qr[:, 1].reshape(1, BLK)

    return pl.pallas_call(
        body,
        grid=(GRID,),
        in_specs=[
            pl.BlockSpec((NC, 1, BLK, D_HID), lambda i: (0, i, 0, 0)),
            pl.BlockSpec((NC, NS, BLK), lambda i: (0, 0, i)),
            pl.BlockSpec((BLK, D_HID), lambda i: (i, 0)),
            pl.BlockSpec((1, D_HID), lambda i: (0, 0)),
            pl.BlockSpec((D_HID, 2), lambda i: (0, 0)),
        ],
        out_specs=[
            pl.BlockSpec((1, BLK), lambda i: (0, i)),
            pl.BlockSpec((1, BLK), lambda i: (0, i)),
        ],
        out_shape=[
            jax.ShapeDtypeStruct((1, N_ACC), jnp.float32),
            jax.ShapeDtypeStruct((1, N_ACC), jnp.float32),
        ],
    )(agg, degp, r1, b1row, w2cat)


def _tc_final(agg2, degp, r2row, b2row):
    def body(a_ref, d_ref, r2_ref, b2_ref, o_ref):
        ssum = jnp.sum(a_ref[...], axis=(0, 1))
        deg = jnp.sum(d_ref[...], axis=(0, 1))
        o = ssum / jnp.maximum(deg, 1.0) + b2_ref[0, 0] + r2_ref[0, :]
        o_ref[...] = o.reshape(BLK, 1)

    return pl.pallas_call(
        body,
        grid=(GRID,),
        in_specs=[
            pl.BlockSpec((NC, NS, BLK), lambda i: (0, 0, i)),
            pl.BlockSpec((NC, NS, BLK), lambda i: (0, 0, i)),
            pl.BlockSpec((1, BLK), lambda i: (0, i)),
            pl.BlockSpec((1, 1), lambda i: (0, 0)),
        ],
        out_specs=pl.BlockSpec((BLK, 1), lambda i: (i, 0)),
        out_shape=jax.ShapeDtypeStruct((N_NODES, 1), jnp.float32),
    )(agg2, degp, r2row, b2row)


def kernel(x, edge_index, W1_l, b1, W1_r, W2_l, b2, W2_r):
    src = edge_index[0].astype(jnp.int32).reshape(NW, N_EDGES // NW)
    dst = edge_index[1].astype(jnp.int32).reshape(NW, N_EDGES // NW)
    pad_src = jnp.broadcast_to(jnp.arange(PAD_PER_TILE, dtype=jnp.int32),
                               (NW, PAD_PER_TILE))
    srcf = jnp.concatenate([src, pad_src], axis=1)
    pad_dst = jnp.broadcast_to(
        N_NODES + jnp.arange(PAD_PER_TILE, dtype=jnp.int32), (NW, PAD_PER_TILE))
    dstf = jnp.concatenate([dst, pad_dst], axis=1)
    src3d = srcf.reshape(NW, STEPS, CHUNK)
    dst3d = dstf.reshape(NW, STEPS, CHUNK)

    xpad = jnp.pad(x, ((0, N_ACC - N_NODES), (0, 0)))
    wcat = jnp.concatenate([W1_l, W1_r], axis=1)
    w2cat = jnp.concatenate([W2_l, W2_r], axis=1)
    zcol = jnp.zeros((N_ACC,), jnp.float32)

    p_tab, r1 = _tc_project1(xpad, wcat)
    degp = _sc_degree(dstf, zcol)
    agg1 = _sc_segment_sum_rows(p_tab, src3d, dst3d,
                                jnp.zeros((ZROWS, D_HID), jnp.float32), degp)
    q_row, r2_row = _tc_middle(agg1, degp, r1, b1.reshape(1, D_HID), w2cat)
    agg2 = _sc_segment_sum_scalar(q_row, srcf, dstf, zcol)
    return _tc_final(agg2, degp, r2_row, b2.reshape(1, 1))
